# Initial kernel scaffold; baseline (speedup 1.0000x reference)
import jax
import jax.numpy as jnp
from jax.experimental import pallas as pl


def kernel(x, edge_index, W, b, betas):
    raise NotImplementedError("write your pallas kernel here")


# speedup vs baseline: 1.1288x; 1.1288x over previous
import functools

import jax
import jax.numpy as jnp
from jax import lax
from jax.experimental import pallas as pl
from jax.experimental.pallas import tpu as pltpu
from jax.experimental.pallas import tpu_sc as plsc

N = 10000
D = 128
E = 320000
NP = 10240
NR = NP // 128
NCORES = 1
NW = NCORES * 16
CHUNK = 96
SUP = 8
NSUP = -(-(E + N) // (NW * SUP * CHUNK))
PER_W = NSUP * SUP * CHUNK
ROWS_PER_TILE = NP // 16

_f32 = jnp.float32
_i32 = jnp.int32


def _sc_conv_body(h_h, xnb_h, src_h, dst_h, zz_h, acc_out, s_out,
                  sidx, didx, xnd, ys, swacc, acc_sh, sem1, sem2):
    cid = lax.axis_index("c")
    sid = lax.axis_index("s")
    wid = sid * NCORES + cid

    pltpu.sync_copy(zz_h.at[pl.ds(0, NR)], swacc)
    row0 = sid * ROWS_PER_TILE
    pltpu.sync_copy(zz_h.at[pl.ds(row0, ROWS_PER_TILE)],
                    acc_sh.at[pl.ds(row0, ROWS_PER_TILE)])
    plsc.subcore_barrier()

    lanes = jnp.arange(16, dtype=_i32)
    seven = jnp.full((16,), 7, _i32)
    low7 = jnp.full((16,), 127, _i32)
    magic = jnp.full((16,), 0x5F3759DF, _i32)
    half = jnp.full((16,), 0.5, _f32)
    threehalf = jnp.full((16,), 1.5, _f32)
    onehot = [lanes == jnp.full((16,), k, _i32) for k in range(16)]

    def sup_body(sj, carry):
        pltpu.sync_copy(src_h.at[wid, sj], sidx)
        pltpu.sync_copy(dst_h.at[wid, sj], didx)

        def chunk_body(c, cc):
            g1 = pltpu.async_copy(xnb_h.at[didx.at[c]], xnd, sem1)
            g2 = pltpu.async_copy(h_h.at[sidx.at[c]], ys, sem2)
            g1.wait()
            g2.wait()

            def group_body(g, gg):
                rows = lanes + g * 16
                dvals = didx[c, pl.ds(g * 16, 16)]

                def dot_step(t, a):
                    dotv, nrmv = a
                    for k in range(8):
                        dv = jnp.full((16,), t * 8 + k, dtype=_i32)
                        xs = plsc.load_gather(ys, [rows, dv])
                        xd = plsc.load_gather(xnd, [rows, dv])
                        dotv = dotv + xs * xd
                        nrmv = nrmv + xs * xs
                    return dotv, nrmv
                dotv, nrmv = lax.fori_loop(
                    0, 16, dot_step,
                    (jnp.zeros((16,), _f32), jnp.zeros((16,), _f32)),
                    unroll=False)

                y = plsc.bitcast(
                    magic - lax.shift_right_logical(
                        plsc.bitcast(nrmv, _i32), jnp.full((16,), 1, _i32)),
                    _f32)
                hn = half * nrmv
                for _ in range(3):
                    y = y * (threehalf - hn * y * y)
                w = jnp.exp(dotv * y)

                dr = dvals >> seven
                dc = dvals & low7
                for k in range(16):
                    plsc.addupdate_scatter(swacc, [dr, dc], w, mask=onehot[k])

                def scale_step(t, ss):
                    for k in range(8):
                        dv = jnp.full((16,), t * 8 + k, dtype=_i32)
                        v = plsc.load_gather(ys, [rows, dv])
                        plsc.store_scatter(ys, [rows, dv], v * w)
                    return ss
                lax.fori_loop(0, 16, scale_step, 0, unroll=False)
                return gg
            lax.fori_loop(0, CHUNK // 16, group_body, 0, unroll=False)
            pltpu.sync_copy(ys, acc_sh.at[didx.at[c]], add=True)
            return cc
        lax.fori_loop(0, SUP, chunk_body, 0, unroll=False)
        return carry

    lax.fori_loop(0, NSUP, sup_body, 0, unroll=False)
    plsc.subcore_barrier()

    pltpu.sync_copy(acc_sh.at[pl.ds(row0, ROWS_PER_TILE)],
                    acc_out.at[cid, pl.ds(row0, ROWS_PER_TILE)])
    pltpu.sync_copy(swacc, s_out.at[cid, sid])


_sc_conv = functools.partial(
    pl.kernel,
    out_type=(jax.ShapeDtypeStruct((NCORES, NP, D), _f32),
              jax.ShapeDtypeStruct((NCORES, 16, NR, 128), _f32)),
    mesh=plsc.VectorSubcoreMesh(core_axis_name="c", subcore_axis_name="s",
                                num_cores=NCORES),
    compiler_params=pltpu.CompilerParams(needs_layout_passes=False),
    scratch_types=[
        pltpu.VMEM((SUP, CHUNK), _i32),
        pltpu.VMEM((SUP, CHUNK), _i32),
        pltpu.VMEM((CHUNK, D), _f32),
        pltpu.VMEM((CHUNK, D), _f32),
        pltpu.VMEM((NR, 128), _f32),
        pltpu.VMEM_SHARED((NP, D), _f32),
        pltpu.SemaphoreType.DMA,
        pltpu.SemaphoreType.DMA,
    ],
)(_sc_conv_body)


def _norm_outputs(hv, beta_ref, y_ref, xnb_ref):
    n2 = jnp.sum(hv * hv, axis=1, keepdims=True)
    rn = 1.0 / jnp.maximum(jnp.sqrt(n2), 1e-12)
    y_ref[...] = hv
    xnb_ref[...] = hv * (rn * beta_ref[0:1, 0:1])


def _prep_body(beta_ref, x_ref, wt_ref, b_ref, res_ref, y_ref, xnb_ref):
    x = x_ref[...]
    res_ref[...] = (jnp.dot(x, wt_ref[...], preferred_element_type=_f32)
                    + b_ref[...])
    _norm_outputs(x, beta_ref, y_ref, xnb_ref)


def _combine(acc_ref, s_ref):
    a = acc_ref[0]
    for c in range(1, NCORES):
        a = a + acc_ref[c]
    i = pl.program_id(0)
    off = (i % 4) * 2
    sblk = s_ref[0, :, pl.ds(off, 2), :]
    for c in range(1, NCORES):
        sblk = sblk + s_ref[c, :, pl.ds(off, 2), :]
    ssum = jnp.sum(sblk, axis=0)
    eye = jnp.eye(128, dtype=_f32)
    cols = [lax.dot_general(eye, ssum[r:r + 1], (((1,), (1,)), ((), ())),
                            preferred_element_type=_f32)
            for r in range(2)]
    s = jnp.concatenate(cols, axis=0)
    return a / s


def _mid_body(beta_ref, rflag_ref, acc_ref, s_ref, y_ref, xnb_ref):
    hv = _combine(acc_ref, s_ref)
    hv = jnp.where(rflag_ref[0:1, 0:1] > 0.0, jnp.maximum(hv, 0.0), hv)
    _norm_outputs(hv, beta_ref, y_ref, xnb_ref)


def _final_body(acc_ref, s_ref, res_ref, out_ref):
    hv = _combine(acc_ref, s_ref)
    out_ref[...] = jnp.maximum(hv + res_ref[...], 0.0)


_ROWS = 256
_GRID = NP // _ROWS

_norm_out_shapes = [
    jax.ShapeDtypeStruct((NP, D), _f32),
    jax.ShapeDtypeStruct((NP, D), _f32),
]
_norm_out_specs = [
    pl.BlockSpec((_ROWS, D), lambda i: (i, 0)),
    pl.BlockSpec((_ROWS, D), lambda i: (i, 0)),
]
_beta_spec = pl.BlockSpec((1, 128), lambda i: (0, 0))
_acc_spec = pl.BlockSpec((NCORES, _ROWS, D), lambda i: (0, i, 0))
_s_spec = pl.BlockSpec((NCORES, 16, 8, 128), lambda i: (0, 0, i // 4, 0))

_tc_prep = pl.pallas_call(
    _prep_body,
    grid=(_GRID,),
    in_specs=[
        _beta_spec,
        pl.BlockSpec((_ROWS, D), lambda i: (i, 0)),
        pl.BlockSpec((D, D), lambda i: (0, 0)),
        pl.BlockSpec((1, D), lambda i: (0, 0)),
    ],
    out_specs=[pl.BlockSpec((_ROWS, D), lambda i: (i, 0))] + _norm_out_specs,
    out_shape=[jax.ShapeDtypeStruct((NP, D), _f32)] + _norm_out_shapes,
)

_tc_mid = pl.pallas_call(
    _mid_body,
    grid=(_GRID,),
    in_specs=[_beta_spec, _beta_spec, _acc_spec, _s_spec],
    out_specs=_norm_out_specs,
    out_shape=_norm_out_shapes,
)

_tc_final = pl.pallas_call(
    _final_body,
    grid=(_GRID,),
    in_specs=[_acc_spec, _s_spec, pl.BlockSpec((_ROWS, D), lambda i: (i, 0))],
    out_specs=pl.BlockSpec((_ROWS, D), lambda i: (i, 0)),
    out_shape=jax.ShapeDtypeStruct((NP, D), _f32),
)


def kernel(x, edge_index, W, b, betas):
    x = x.astype(_f32)
    xp = jnp.zeros((NP, D), _f32).at[:N].set(x)

    loops = jnp.arange(N, dtype=_i32)
    src = jnp.concatenate([edge_index[0].astype(_i32), loops])
    dst = jnp.concatenate([edge_index[1].astype(_i32), loops])
    pad = NW * PER_W - src.shape[0]
    padv = jnp.full((pad,), N, dtype=_i32)
    srcp = jnp.concatenate([src, padv]).reshape(NW, NSUP, SUP, CHUNK)
    dstp = jnp.concatenate([dst, padv]).reshape(NW, NSUP, SUP, CHUNK)

    wt = W.astype(_f32).T
    b2 = b.astype(_f32).reshape(1, D)
    zz = jnp.zeros((NP, D), _f32)
    brow = [jnp.broadcast_to(betas[i].astype(_f32), (1, 128)) for i in range(3)]

    res, y, xnb = _tc_prep(brow[0], xp, wt, b2)

    def layer(carry, xs):
        y, xnb, _, _ = carry
        brow_next, rflag = xs
        acc, s = _sc_conv(y, xnb, srcp, dstp, zz)
        y2, xnb2 = _tc_mid(brow_next, rflag, acc, s)
        return (y2, xnb2, acc, s), None

    ones = jnp.ones((1, 128), _f32)
    brow_next = jnp.stack([brow[1], brow[2], brow[2]])
    rflags = jnp.stack([ones, 0.0 * ones, 0.0 * ones])
    acc0 = jnp.zeros((NCORES, NP, D), _f32)
    s0 = jnp.zeros((NCORES, 16, NR, 128), _f32)
    carry0 = (y, xnb, acc0, s0)
    (_, _, acc, s), _ = lax.scan(layer, carry0, (brow_next, rflags))
    out = _tc_final(acc, s, res)
    return out[:N]


# speedup vs baseline: 1.1657x; 1.0327x over previous
import functools

import jax
import jax.numpy as jnp
from jax import lax
from jax.experimental import pallas as pl
from jax.experimental.pallas import tpu as pltpu
from jax.experimental.pallas import tpu_sc as plsc

N = 10000
D = 128
E = 320000
NP = 10240
NR = NP // 128
NCORES = 1
NW = NCORES * 16
CHUNK = 64
SUP = 8
NSUP = -(-(E + N) // (NW * SUP * CHUNK))
PER_W = NSUP * SUP * CHUNK
ROWS_PER_TILE = NP // 16

_f32 = jnp.float32
_i32 = jnp.int32


def _sc_conv_body(h_h, xnb_h, src_h, dst_h, zz_h, acc_out, s_out,
                  sidx, didx, xnda, ysa, xndb, ysb, swacc, acc_sh,
                  sem1, sem2, sem3, sem4):
    cid = lax.axis_index("c")
    sid = lax.axis_index("s")
    wid = sid * NCORES + cid

    pltpu.sync_copy(zz_h.at[pl.ds(0, NR)], swacc)
    row0 = sid * ROWS_PER_TILE
    pltpu.sync_copy(zz_h.at[pl.ds(row0, ROWS_PER_TILE)],
                    acc_sh.at[pl.ds(row0, ROWS_PER_TILE)])
    plsc.subcore_barrier()

    lanes = jnp.arange(16, dtype=_i32)
    seven = jnp.full((16,), 7, _i32)
    low7 = jnp.full((16,), 127, _i32)
    magic = jnp.full((16,), 0x5F3759DF, _i32)
    half = jnp.full((16,), 0.5, _f32)
    threehalf = jnp.full((16,), 1.5, _f32)
    onehot = [lanes == jnp.full((16,), k, _i32) for k in range(16)]

    def compute_chunk(c, xnd, ys):
        def group_body(g, gg):
            rows = lanes + g * 16
            dvals = didx[c, pl.ds(g * 16, 16)]

            def dot_step(t, a):
                dotv, nrmv = a
                for k in range(8):
                    dv = jnp.full((16,), t * 8 + k, dtype=_i32)
                    xs = plsc.load_gather(ys, [rows, dv])
                    xd = plsc.load_gather(xnd, [rows, dv])
                    dotv = dotv + xs * xd
                    nrmv = nrmv + xs * xs
                return dotv, nrmv
            dotv, nrmv = lax.fori_loop(
                0, 16, dot_step,
                (jnp.zeros((16,), _f32), jnp.zeros((16,), _f32)),
                unroll=2)

            y = plsc.bitcast(
                magic - lax.shift_right_logical(
                    plsc.bitcast(nrmv, _i32), jnp.full((16,), 1, _i32)),
                _f32)
            hn = half * nrmv
            for _ in range(3):
                y = y * (threehalf - hn * y * y)
            w = jnp.exp(dotv * y)

            dr = dvals >> seven
            dc = dvals & low7
            for k in range(16):
                plsc.addupdate_scatter(swacc, [dr, dc], w, mask=onehot[k])

            def scale_step(t, ss):
                for k in range(8):
                    dv = jnp.full((16,), t * 8 + k, dtype=_i32)
                    v = plsc.load_gather(ys, [rows, dv])
                    plsc.store_scatter(ys, [rows, dv], v * w)
                return ss
            lax.fori_loop(0, 16, scale_step, 0, unroll=2)
            return gg
        lax.fori_loop(0, CHUNK // 16, group_body, 0, unroll=False)
        pltpu.sync_copy(ys, acc_sh.at[didx.at[c]], add=True)

    def start_chunk(c, xnd, ys, s1, s2):
        pltpu.async_copy(xnb_h.at[didx.at[c]], xnd, s1)
        pltpu.async_copy(h_h.at[sidx.at[c]], ys, s2)

    def wait_chunk(xnd, ys, s1, s2):
        pltpu.make_async_copy(xnb_h.at[didx.at[0]], xnd, s1).wait()
        pltpu.make_async_copy(h_h.at[sidx.at[0]], ys, s2).wait()

    def sup_body(sj, carry):
        pltpu.sync_copy(src_h.at[wid, sj], sidx)
        pltpu.sync_copy(dst_h.at[wid, sj], didx)
        start_chunk(0, xnda, ysa, sem1, sem2)

        def pair_body(pp, cc):
            c0 = 2 * pp
            wait_chunk(xnda, ysa, sem1, sem2)
            start_chunk(c0 + 1, xndb, ysb, sem3, sem4)
            compute_chunk(c0, xnda, ysa)
            wait_chunk(xndb, ysb, sem3, sem4)
            start_chunk(c0 + 2, xnda, ysa, sem1, sem2)
            compute_chunk(c0 + 1, xndb, ysb)
            return cc
        lax.fori_loop(0, SUP // 2 - 1, pair_body, 0, unroll=False)

        wait_chunk(xnda, ysa, sem1, sem2)
        start_chunk(SUP - 1, xndb, ysb, sem3, sem4)
        compute_chunk(SUP - 2, xnda, ysa)
        wait_chunk(xndb, ysb, sem3, sem4)
        compute_chunk(SUP - 1, xndb, ysb)
        return carry

    lax.fori_loop(0, NSUP, sup_body, 0, unroll=False)
    plsc.subcore_barrier()

    pltpu.sync_copy(acc_sh.at[pl.ds(row0, ROWS_PER_TILE)],
                    acc_out.at[cid, pl.ds(row0, ROWS_PER_TILE)])
    pltpu.sync_copy(swacc, s_out.at[cid, sid])


_sc_conv = functools.partial(
    pl.kernel,
    out_type=(jax.ShapeDtypeStruct((NCORES, NP, D), _f32),
              jax.ShapeDtypeStruct((NCORES, 16, NR, 128), _f32)),
    mesh=plsc.VectorSubcoreMesh(core_axis_name="c", subcore_axis_name="s",
                                num_cores=NCORES),
    compiler_params=pltpu.CompilerParams(needs_layout_passes=False),
    scratch_types=[
        pltpu.VMEM((SUP, CHUNK), _i32),
        pltpu.VMEM((SUP, CHUNK), _i32),
        pltpu.VMEM((CHUNK, D), _f32),
        pltpu.VMEM((CHUNK, D), _f32),
        pltpu.VMEM((CHUNK, D), _f32),
        pltpu.VMEM((CHUNK, D), _f32),
        pltpu.VMEM((NR, 128), _f32),
        pltpu.VMEM_SHARED((NP, D), _f32),
        pltpu.SemaphoreType.DMA,
        pltpu.SemaphoreType.DMA,
        pltpu.SemaphoreType.DMA,
        pltpu.SemaphoreType.DMA,
    ],
)(_sc_conv_body)


def _norm_outputs(hv, beta_ref, y_ref, xnb_ref):
    n2 = jnp.sum(hv * hv, axis=1, keepdims=True)
    rn = 1.0 / jnp.maximum(jnp.sqrt(n2), 1e-12)
    y_ref[...] = hv
    xnb_ref[...] = hv * (rn * beta_ref[0:1, 0:1])


def _prep_body(beta_ref, x_ref, wt_ref, b_ref, res_ref, y_ref, xnb_ref):
    x = x_ref[...]
    res_ref[...] = (jnp.dot(x, wt_ref[...], preferred_element_type=_f32)
                    + b_ref[...])
    _norm_outputs(x, beta_ref, y_ref, xnb_ref)


def _combine(acc_ref, s_ref):
    a = acc_ref[0]
    for c in range(1, NCORES):
        a = a + acc_ref[c]
    i = pl.program_id(0)
    off = (i % 4) * 2
    sblk = s_ref[0, :, pl.ds(off, 2), :]
    for c in range(1, NCORES):
        sblk = sblk + s_ref[c, :, pl.ds(off, 2), :]
    ssum = jnp.sum(sblk, axis=0)
    eye = jnp.eye(128, dtype=_f32)
    cols = [lax.dot_general(eye, ssum[r:r + 1], (((1,), (1,)), ((), ())),
                            preferred_element_type=_f32)
            for r in range(2)]
    s = jnp.concatenate(cols, axis=0)
    return a / s


def _mid_body(beta_ref, rflag_ref, acc_ref, s_ref, y_ref, xnb_ref):
    hv = _combine(acc_ref, s_ref)
    hv = jnp.where(rflag_ref[0:1, 0:1] > 0.0, jnp.maximum(hv, 0.0), hv)
    _norm_outputs(hv, beta_ref, y_ref, xnb_ref)


def _final_body(acc_ref, s_ref, res_ref, out_ref):
    hv = _combine(acc_ref, s_ref)
    out_ref[...] = jnp.maximum(hv + res_ref[...], 0.0)


_ROWS = 256
_GRID = NP // _ROWS

_norm_out_shapes = [
    jax.ShapeDtypeStruct((NP, D), _f32),
    jax.ShapeDtypeStruct((NP, D), _f32),
]
_norm_out_specs = [
    pl.BlockSpec((_ROWS, D), lambda i: (i, 0)),
    pl.BlockSpec((_ROWS, D), lambda i: (i, 0)),
]
_beta_spec = pl.BlockSpec((1, 128), lambda i: (0, 0))
_acc_spec = pl.BlockSpec((NCORES, _ROWS, D), lambda i: (0, i, 0))
_s_spec = pl.BlockSpec((NCORES, 16, 8, 128), lambda i: (0, 0, i // 4, 0))

_tc_prep = pl.pallas_call(
    _prep_body,
    grid=(_GRID,),
    in_specs=[
        _beta_spec,
        pl.BlockSpec((_ROWS, D), lambda i: (i, 0)),
        pl.BlockSpec((D, D), lambda i: (0, 0)),
        pl.BlockSpec((1, D), lambda i: (0, 0)),
    ],
    out_specs=[pl.BlockSpec((_ROWS, D), lambda i: (i, 0))] + _norm_out_specs,
    out_shape=[jax.ShapeDtypeStruct((NP, D), _f32)] + _norm_out_shapes,
)

_tc_mid = pl.pallas_call(
    _mid_body,
    grid=(_GRID,),
    in_specs=[_beta_spec, _beta_spec, _acc_spec, _s_spec],
    out_specs=_norm_out_specs,
    out_shape=_norm_out_shapes,
)

_tc_final = pl.pallas_call(
    _final_body,
    grid=(_GRID,),
    in_specs=[_acc_spec, _s_spec, pl.BlockSpec((_ROWS, D), lambda i: (i, 0))],
    out_specs=pl.BlockSpec((_ROWS, D), lambda i: (i, 0)),
    out_shape=jax.ShapeDtypeStruct((NP, D), _f32),
)


def kernel(x, edge_index, W, b, betas):
    x = x.astype(_f32)
    xp = jnp.zeros((NP, D), _f32).at[:N].set(x)

    loops = jnp.arange(N, dtype=_i32)
    src = jnp.concatenate([edge_index[0].astype(_i32), loops])
    dst = jnp.concatenate([edge_index[1].astype(_i32), loops])
    pad = NW * PER_W - src.shape[0]
    padv = jnp.full((pad,), N, dtype=_i32)
    srcp = jnp.concatenate([src, padv]).reshape(NW, NSUP, SUP, CHUNK)
    dstp = jnp.concatenate([dst, padv]).reshape(NW, NSUP, SUP, CHUNK)

    wt = W.astype(_f32).T
    b2 = b.astype(_f32).reshape(1, D)
    zz = jnp.zeros((NP, D), _f32)
    brow = [jnp.broadcast_to(betas[i].astype(_f32), (1, 128)) for i in range(3)]

    res, y, xnb = _tc_prep(brow[0], xp, wt, b2)

    def layer(carry, xs):
        y, xnb, _, _ = carry
        brow_next, rflag = xs
        acc, s = _sc_conv(y, xnb, srcp, dstp, zz)
        y2, xnb2 = _tc_mid(brow_next, rflag, acc, s)
        return (y2, xnb2, acc, s), None

    ones = jnp.ones((1, 128), _f32)
    brow_next = jnp.stack([brow[1], brow[2], brow[2]])
    rflags = jnp.stack([ones, 0.0 * ones, 0.0 * ones])
    acc0 = jnp.zeros((NCORES, NP, D), _f32)
    s0 = jnp.zeros((NCORES, 16, NR, 128), _f32)
    carry0 = (y, xnb, acc0, s0)
    (_, _, acc, s), _ = lax.scan(layer, carry0, (brow_next, rflags))
    out = _tc_final(acc, s, res)
    return out[:N]


# speedup vs baseline: 4.9880x; 4.2790x over previous
import functools

import jax
import jax.numpy as jnp
from jax import lax
from jax.experimental import pallas as pl
from jax.experimental.pallas import tpu as pltpu
from jax.experimental.pallas import tpu_sc as plsc

N = 10000
D = 128
E = 320000
NP = 10240
NR = NP // 128
NCORES = 1
NW = NCORES * 16
CHUNK = 64
SUP = 8
NSUP = -(-(E + N) // (NW * SUP * CHUNK))
PER_W = NSUP * SUP * CHUNK
ROWS_PER_TILE = NP // 16

_f32 = jnp.float32
_i32 = jnp.int32


def _sc_conv_body(h_h, xnb_h, src_h, dst_h, zz_h, acc_out, s_out,
                  sidx, didx, xnda, ysa, xndb, ysb, swacc, acc_sh,
                  sem1, sem2, sem3, sem4):
    cid = lax.axis_index("c")
    sid = lax.axis_index("s")
    wid = sid * NCORES + cid

    pltpu.sync_copy(zz_h.at[pl.ds(0, NR)], swacc)
    row0 = sid * ROWS_PER_TILE
    pltpu.sync_copy(zz_h.at[pl.ds(row0, ROWS_PER_TILE)],
                    acc_sh.at[pl.ds(row0, ROWS_PER_TILE)])
    plsc.subcore_barrier()

    lanes = jnp.arange(16, dtype=_i32)
    seven = jnp.full((16,), 7, _i32)
    low7 = jnp.full((16,), 127, _i32)
    magic = jnp.full((16,), 0x5F3759DF, _i32)
    half = jnp.full((16,), 0.5, _f32)
    threehalf = jnp.full((16,), 1.5, _f32)
    onehot = [lanes == jnp.full((16,), k, _i32) for k in range(16)]

    def compute_chunk(c, xnd, ys):
        def group_body(g, gg):
            rows = lanes + g * 16
            dvals = didx[c, pl.ds(g * 16, 16)]

            def dot_step(t, a):
                dotv, nrmv = a
                for k in range(8):
                    dv = (jnp.full((16,), t * 8 + k, dtype=_i32)
                          + lanes) & low7
                    xs = plsc.load_gather(ys, [rows, dv])
                    xd = plsc.load_gather(xnd, [rows, dv])
                    dotv = dotv + xs * xd
                    nrmv = nrmv + xs * xs
                return dotv, nrmv
            dotv, nrmv = lax.fori_loop(
                0, 16, dot_step,
                (jnp.zeros((16,), _f32), jnp.zeros((16,), _f32)),
                unroll=2)

            y = plsc.bitcast(
                magic - lax.shift_right_logical(
                    plsc.bitcast(nrmv, _i32), jnp.full((16,), 1, _i32)),
                _f32)
            hn = half * nrmv
            for _ in range(3):
                y = y * (threehalf - hn * y * y)
            w = jnp.exp(dotv * y)

            dr = dvals >> seven
            dc = dvals & low7
            for k in range(16):
                plsc.addupdate_scatter(swacc, [dr, dc], w, mask=onehot[k])

            def scale_step(t, ss):
                for k in range(8):
                    dv = (jnp.full((16,), t * 8 + k, dtype=_i32)
                          + lanes) & low7
                    v = plsc.load_gather(ys, [rows, dv])
                    plsc.store_scatter(ys, [rows, dv], v * w)
                return ss
            lax.fori_loop(0, 16, scale_step, 0, unroll=2)
            return gg
        lax.fori_loop(0, CHUNK // 16, group_body, 0, unroll=False)
        pltpu.sync_copy(ys, acc_sh.at[didx.at[c]], add=True)

    def start_chunk(c, xnd, ys, s1, s2):
        pltpu.async_copy(xnb_h.at[didx.at[c]], xnd, s1)
        pltpu.async_copy(h_h.at[sidx.at[c]], ys, s2)

    def wait_chunk(xnd, ys, s1, s2):
        pltpu.make_async_copy(xnb_h.at[didx.at[0]], xnd, s1).wait()
        pltpu.make_async_copy(h_h.at[sidx.at[0]], ys, s2).wait()

    def sup_body(sj, carry):
        pltpu.sync_copy(src_h.at[wid, sj], sidx)
        pltpu.sync_copy(dst_h.at[wid, sj], didx)
        start_chunk(0, xnda, ysa, sem1, sem2)

        def pair_body(pp, cc):
            c0 = 2 * pp
            wait_chunk(xnda, ysa, sem1, sem2)
            start_chunk(c0 + 1, xndb, ysb, sem3, sem4)
            compute_chunk(c0, xnda, ysa)
            wait_chunk(xndb, ysb, sem3, sem4)
            start_chunk(c0 + 2, xnda, ysa, sem1, sem2)
            compute_chunk(c0 + 1, xndb, ysb)
            return cc
        lax.fori_loop(0, SUP // 2 - 1, pair_body, 0, unroll=False)

        wait_chunk(xnda, ysa, sem1, sem2)
        start_chunk(SUP - 1, xndb, ysb, sem3, sem4)
        compute_chunk(SUP - 2, xnda, ysa)
        wait_chunk(xndb, ysb, sem3, sem4)
        compute_chunk(SUP - 1, xndb, ysb)
        return carry

    lax.fori_loop(0, NSUP, sup_body, 0, unroll=False)
    plsc.subcore_barrier()

    pltpu.sync_copy(acc_sh.at[pl.ds(row0, ROWS_PER_TILE)],
                    acc_out.at[cid, pl.ds(row0, ROWS_PER_TILE)])
    pltpu.sync_copy(swacc, s_out.at[cid, sid])


_sc_conv = functools.partial(
    pl.kernel,
    out_type=(jax.ShapeDtypeStruct((NCORES, NP, D), _f32),
              jax.ShapeDtypeStruct((NCORES, 16, NR, 128), _f32)),
    mesh=plsc.VectorSubcoreMesh(core_axis_name="c", subcore_axis_name="s",
                                num_cores=NCORES),
    compiler_params=pltpu.CompilerParams(needs_layout_passes=False),
    scratch_types=[
        pltpu.VMEM((SUP, CHUNK), _i32),
        pltpu.VMEM((SUP, CHUNK), _i32),
        pltpu.VMEM((CHUNK, D), _f32),
        pltpu.VMEM((CHUNK, D), _f32),
        pltpu.VMEM((CHUNK, D), _f32),
        pltpu.VMEM((CHUNK, D), _f32),
        pltpu.VMEM((NR, 128), _f32),
        pltpu.VMEM_SHARED((NP, D), _f32),
        pltpu.SemaphoreType.DMA,
        pltpu.SemaphoreType.DMA,
        pltpu.SemaphoreType.DMA,
        pltpu.SemaphoreType.DMA,
    ],
)(_sc_conv_body)


def _norm_outputs(hv, beta_ref, y_ref, xnb_ref):
    n2 = jnp.sum(hv * hv, axis=1, keepdims=True)
    rn = 1.0 / jnp.maximum(jnp.sqrt(n2), 1e-12)
    y_ref[...] = hv
    xnb_ref[...] = hv * (rn * beta_ref[0:1, 0:1])


def _prep_body(beta_ref, x_ref, wt_ref, b_ref, res_ref, y_ref, xnb_ref):
    x = x_ref[...]
    res_ref[...] = (jnp.dot(x, wt_ref[...], preferred_element_type=_f32)
                    + b_ref[...])
    _norm_outputs(x, beta_ref, y_ref, xnb_ref)


def _combine(acc_ref, s_ref):
    a = acc_ref[0]
    for c in range(1, NCORES):
        a = a + acc_ref[c]
    i = pl.program_id(0)
    off = (i % 4) * 2
    sblk = s_ref[0, :, pl.ds(off, 2), :]
    for c in range(1, NCORES):
        sblk = sblk + s_ref[c, :, pl.ds(off, 2), :]
    ssum = jnp.sum(sblk, axis=0)
    eye = jnp.eye(128, dtype=_f32)
    cols = [lax.dot_general(eye, ssum[r:r + 1], (((1,), (1,)), ((), ())),
                            preferred_element_type=_f32)
            for r in range(2)]
    s = jnp.concatenate(cols, axis=0)
    return a / s


def _mid_body(beta_ref, rflag_ref, acc_ref, s_ref, y_ref, xnb_ref):
    hv = _combine(acc_ref, s_ref)
    hv = jnp.where(rflag_ref[0:1, 0:1] > 0.0, jnp.maximum(hv, 0.0), hv)
    _norm_outputs(hv, beta_ref, y_ref, xnb_ref)


def _final_body(acc_ref, s_ref, res_ref, out_ref):
    hv = _combine(acc_ref, s_ref)
    out_ref[...] = jnp.maximum(hv + res_ref[...], 0.0)


_ROWS = 256
_GRID = NP // _ROWS

_norm_out_shapes = [
    jax.ShapeDtypeStruct((NP, D), _f32),
    jax.ShapeDtypeStruct((NP, D), _f32),
]
_norm_out_specs = [
    pl.BlockSpec((_ROWS, D), lambda i: (i, 0)),
    pl.BlockSpec((_ROWS, D), lambda i: (i, 0)),
]
_beta_spec = pl.BlockSpec((1, 128), lambda i: (0, 0))
_acc_spec = pl.BlockSpec((NCORES, _ROWS, D), lambda i: (0, i, 0))
_s_spec = pl.BlockSpec((NCORES, 16, 8, 128), lambda i: (0, 0, i // 4, 0))

_tc_prep = pl.pallas_call(
    _prep_body,
    grid=(_GRID,),
    in_specs=[
        _beta_spec,
        pl.BlockSpec((_ROWS, D), lambda i: (i, 0)),
        pl.BlockSpec((D, D), lambda i: (0, 0)),
        pl.BlockSpec((1, D), lambda i: (0, 0)),
    ],
    out_specs=[pl.BlockSpec((_ROWS, D), lambda i: (i, 0))] + _norm_out_specs,
    out_shape=[jax.ShapeDtypeStruct((NP, D), _f32)] + _norm_out_shapes,
)

_tc_mid = pl.pallas_call(
    _mid_body,
    grid=(_GRID,),
    in_specs=[_beta_spec, _beta_spec, _acc_spec, _s_spec],
    out_specs=_norm_out_specs,
    out_shape=_norm_out_shapes,
)

_tc_final = pl.pallas_call(
    _final_body,
    grid=(_GRID,),
    in_specs=[_acc_spec, _s_spec, pl.BlockSpec((_ROWS, D), lambda i: (i, 0))],
    out_specs=pl.BlockSpec((_ROWS, D), lambda i: (i, 0)),
    out_shape=jax.ShapeDtypeStruct((NP, D), _f32),
)


def kernel(x, edge_index, W, b, betas):
    x = x.astype(_f32)
    xp = jnp.zeros((NP, D), _f32).at[:N].set(x)

    loops = jnp.arange(N, dtype=_i32)
    src = jnp.concatenate([edge_index[0].astype(_i32), loops])
    dst = jnp.concatenate([edge_index[1].astype(_i32), loops])
    pad = NW * PER_W - src.shape[0]
    padv = jnp.full((pad,), N, dtype=_i32)
    srcp = jnp.concatenate([src, padv]).reshape(NW, NSUP, SUP, CHUNK)
    dstp = jnp.concatenate([dst, padv]).reshape(NW, NSUP, SUP, CHUNK)

    wt = W.astype(_f32).T
    b2 = b.astype(_f32).reshape(1, D)
    zz = jnp.zeros((NP, D), _f32)
    brow = [jnp.broadcast_to(betas[i].astype(_f32), (1, 128)) for i in range(3)]

    res, y, xnb = _tc_prep(brow[0], xp, wt, b2)

    def layer(carry, xs):
        y, xnb, _, _ = carry
        brow_next, rflag = xs
        acc, s = _sc_conv(y, xnb, srcp, dstp, zz)
        y2, xnb2 = _tc_mid(brow_next, rflag, acc, s)
        return (y2, xnb2, acc, s), None

    ones = jnp.ones((1, 128), _f32)
    brow_next = jnp.stack([brow[1], brow[2], brow[2]])
    rflags = jnp.stack([ones, 0.0 * ones, 0.0 * ones])
    acc0 = jnp.zeros((NCORES, NP, D), _f32)
    s0 = jnp.zeros((NCORES, 16, NR, 128), _f32)
    carry0 = (y, xnb, acc0, s0)
    (_, _, acc, s), _ = lax.scan(layer, carry0, (brow_next, rflags))
    out = _tc_final(acc, s, res)
    return out[:N]
